# baseline (speedup 1.0000x reference)
import functools

import jax
import jax.numpy as jnp
import numpy as np
from jax.experimental import pallas as pl


def _focal_kernel(cls_ref, reg_ref, anch_ref, ann_ref, annT_ref, mT_ref,
                  out_ref):
    j = pl.program_id(1)

    x = cls_ref[0]
    rT = jnp.swapaxes(reg_ref[0], 0, 1)
    anchT = jnp.swapaxes(anch_ref[0], 0, 1)
    ann = ann_ref[0]
    annT = annT_ref[0]
    MT = mT_ref[...]

    BLK, C = x.shape
    N = ann.shape[0]

    def col(i):
        return ann[:, i:i + 1]

    xmin = jnp.minimum(jnp.minimum(col(0), col(2)), jnp.minimum(col(4), col(6)))
    xmax = jnp.maximum(jnp.maximum(col(0), col(2)), jnp.maximum(col(4), col(6)))
    ymin = jnp.minimum(jnp.minimum(col(1), col(3)), jnp.minimum(col(5), col(7)))
    ymax = jnp.maximum(jnp.maximum(col(1), col(3)), jnp.maximum(col(5), col(7)))
    xmin2 = jnp.minimum(jnp.minimum(col(8), col(10)), jnp.minimum(col(12), col(14)))
    xmax2 = jnp.maximum(jnp.maximum(col(8), col(10)), jnp.maximum(col(12), col(14)))
    ymin2 = jnp.minimum(jnp.minimum(col(9), col(11)), jnp.minimum(col(13), col(15)))
    ymax2 = jnp.maximum(jnp.maximum(col(9), col(11)), jnp.maximum(col(13), col(15)))
    bx1 = jnp.minimum(xmin, xmin2)
    by1 = jnp.minimum(ymin, ymin2)
    bx2 = jnp.maximum(xmax, xmax2)
    by2 = jnp.maximum(ymax, ymax2)
    barea = (bx2 - bx1) * (by2 - by1)

    ax1 = anchT[0:1, :]
    ay1 = anchT[1:2, :]
    ax2 = anchT[2:3, :]
    ay2 = anchT[3:4, :]
    aw = ax2 - ax1
    ah = ay2 - ay1
    acx = ax1 + 0.5 * aw
    acy = ay1 + 0.5 * ah
    aarea = aw * ah

    iw = jnp.clip(jnp.minimum(ax2, bx2) - jnp.maximum(ax1, bx1), 0.0)
    ih = jnp.clip(jnp.minimum(ay2, by2) - jnp.maximum(ay1, by1), 0.0)
    inter = iw * ih
    ua = jnp.clip(aarea + barea - inter, 1e-8)
    iou = inter / ua

    iou_max = jnp.max(iou, axis=0, keepdims=True)
    idx = jax.lax.broadcasted_iota(jnp.int32, (N, BLK), 0)
    arg = jnp.min(jnp.where(iou == iou_max, idx, N), axis=0, keepdims=True)
    onehotT = (idx == arg).astype(jnp.float32)

    posf = (iou_max >= 0.5).astype(jnp.float32)
    negf = (iou_max < 0.4).astype(jnp.float32)
    num_pos = jnp.sum(posf)

    p = jnp.clip(x, 1e-4, 1.0 - 1e-4)
    negterm = (p * p) * jnp.log(1.0 - p)
    w_row = negf + posf
    t1 = jax.lax.dot_general(w_row, negterm, (((1,), (0,)), ((), ())),
                             preferred_element_type=jnp.float32)
    cls_main = -0.75 * jnp.sum(t1)

    cidx = jax.lax.broadcasted_iota(jnp.int32, (C, N), 0)
    ohAT = (cidx == annT[20:21, :].astype(jnp.int32)).astype(jnp.float32)
    G = jax.lax.dot_general(p, ohAT, (((1,), (0,)), ((), ())),
                            preferred_element_type=jnp.float32)
    GT = jnp.swapaxes(G, 0, 1)
    pc = jnp.sum(onehotT * GT, axis=0, keepdims=True)
    fpos = 0.25 * (1.0 - pc) * (1.0 - pc) * (-jnp.log(pc))
    fneg = 0.75 * pc * pc * (-jnp.log(1.0 - pc))
    corr = jnp.sum(posf * (fpos - fneg))
    cls_blk = cls_main + corr

    predsT = jax.lax.dot_general(MT, rT, (((1,), (0,)), ((), ())),
                                 preferred_element_type=jnp.float32)
    tT = jax.lax.dot_general(annT[0:20, :], onehotT, (((1,), (0,)), ((), ())),
                             preferred_element_type=jnp.float32)
    rowi = jax.lax.broadcasted_iota(jnp.int32, (20, BLK), 0)
    is_x = (rowi % 2) == 0
    inv_aw = 1.0 / aw
    inv_ah = 1.0 / ah
    t_norm = jnp.where(is_x, (tT - acx) * inv_aw, (tT - acy) * inv_ah)
    diff = jnp.abs(t_norm - predsT)
    rl = jnp.where(diff <= 1.0 / 9.0, 4.5 * diff * diff, diff - 0.5 / 9.0)
    reg_blk = jnp.sum(rl * posf)

    lane = jax.lax.broadcasted_iota(jnp.int32, (1, 1, 128), 2)
    vec = (jnp.where(lane == 0, cls_blk, 0.0)
           + jnp.where(lane == 1, reg_blk, 0.0)
           + jnp.where(lane == 2, num_pos, 0.0))

    @pl.when(j == 0)
    def _init():
        out_ref[...] = vec

    @pl.when(j != 0)
    def _accum():
        out_ref[...] = out_ref[...] + vec


def _pred_matrix_t() -> np.ndarray:
    m = np.zeros((12, 20), np.float32)
    for pt in range(8):
        s1 = 1.0 if pt & 1 else -1.0
        s2 = 1.0 if pt & 2 else -1.0
        s3 = 1.0 if pt & 4 else -1.0
        m[0, 2 * pt] = 1.0
        m[2, 2 * pt] = s1
        m[4, 2 * pt] = s2
        m[6, 2 * pt] = s3
        m[1, 2 * pt + 1] = 1.0
        m[3, 2 * pt + 1] = s1
        m[5, 2 * pt + 1] = s2
        m[7, 2 * pt + 1] = s3
    for k in range(4):
        m[8 + k, 16 + k] = 1.0
    return m.T.copy()


@functools.partial(jax.jit, static_argnames=("blk",))
def _run(classifications, regressions, anchors, annotations, blk):
    B, A, C = classifications.shape
    N = annotations.shape[1]
    nblk = A // blk
    annT = jnp.swapaxes(annotations, 1, 2)
    mT = jnp.asarray(_pred_matrix_t())

    out = pl.pallas_call(
        _focal_kernel,
        grid=(B, nblk),
        in_specs=[
            pl.BlockSpec((1, blk, C), lambda b, j: (b, j, 0)),
            pl.BlockSpec((1, blk, 12), lambda b, j: (b, j, 0)),
            pl.BlockSpec((1, blk, 4), lambda b, j: (0, j, 0)),
            pl.BlockSpec((1, N, 21), lambda b, j: (b, 0, 0)),
            pl.BlockSpec((1, 21, N), lambda b, j: (b, 0, 0)),
            pl.BlockSpec((20, 12), lambda b, j: (0, 0)),
        ],
        out_specs=pl.BlockSpec((1, 1, 128), lambda b, j: (b, 0, 0)),
        out_shape=jax.ShapeDtypeStruct((B, 1, 128), jnp.float32),
    )(classifications, regressions, anchors, annotations, annT, mT)

    cls_sum = out[:, 0, 0]
    reg_sum = out[:, 0, 1]
    npos = out[:, 0, 2]
    cls_total = cls_sum / jnp.maximum(npos, 1.0)
    reg_total = jnp.where(npos > 0.0,
                          reg_sum / (jnp.maximum(npos, 1.0) * 20.0), 0.0)
    return (jnp.mean(cls_total)[None], jnp.mean(reg_total)[None])


def kernel(classifications, regressions, anchors, annotations):
    A = classifications.shape[1]
    blk = 4000 if A % 4000 == 0 else A
    return _run(classifications, regressions, anchors, annotations, blk)


# speedup vs baseline: 1.0003x; 1.0003x over previous
import functools

import jax
import jax.numpy as jnp
import numpy as np
from jax.experimental import pallas as pl


def _focal_kernel(cls_ref, reg_ref, anch_ref, ann_ref, annT_ref, mT_ref,
                  out_ref):
    j = pl.program_id(1)

    x = cls_ref[0]
    r = reg_ref[0]
    anch = anch_ref[0]
    eye4 = (jax.lax.broadcasted_iota(jnp.int32, (4, 4), 0)
            == jax.lax.broadcasted_iota(jnp.int32, (4, 4), 1)).astype(jnp.float32)
    anchT = jax.lax.dot_general(eye4, anch, (((1,), (1,)), ((), ())),
                                preferred_element_type=jnp.float32)
    ann = ann_ref[0]
    annT = annT_ref[0]
    MT = mT_ref[...]

    BLK, C = x.shape
    N = ann.shape[0]

    def col(i):
        return ann[:, i:i + 1]

    xmin = jnp.minimum(jnp.minimum(col(0), col(2)), jnp.minimum(col(4), col(6)))
    xmax = jnp.maximum(jnp.maximum(col(0), col(2)), jnp.maximum(col(4), col(6)))
    ymin = jnp.minimum(jnp.minimum(col(1), col(3)), jnp.minimum(col(5), col(7)))
    ymax = jnp.maximum(jnp.maximum(col(1), col(3)), jnp.maximum(col(5), col(7)))
    xmin2 = jnp.minimum(jnp.minimum(col(8), col(10)), jnp.minimum(col(12), col(14)))
    xmax2 = jnp.maximum(jnp.maximum(col(8), col(10)), jnp.maximum(col(12), col(14)))
    ymin2 = jnp.minimum(jnp.minimum(col(9), col(11)), jnp.minimum(col(13), col(15)))
    ymax2 = jnp.maximum(jnp.maximum(col(9), col(11)), jnp.maximum(col(13), col(15)))
    bx1 = jnp.minimum(xmin, xmin2)
    by1 = jnp.minimum(ymin, ymin2)
    bx2 = jnp.maximum(xmax, xmax2)
    by2 = jnp.maximum(ymax, ymax2)
    barea = (bx2 - bx1) * (by2 - by1)

    ax1 = anchT[0:1, :]
    ay1 = anchT[1:2, :]
    ax2 = anchT[2:3, :]
    ay2 = anchT[3:4, :]
    aw = ax2 - ax1
    ah = ay2 - ay1
    acx = ax1 + 0.5 * aw
    acy = ay1 + 0.5 * ah
    aarea = aw * ah

    iw = jnp.clip(jnp.minimum(ax2, bx2) - jnp.maximum(ax1, bx1), 0.0)
    ih = jnp.clip(jnp.minimum(ay2, by2) - jnp.maximum(ay1, by1), 0.0)
    inter = iw * ih
    ua = jnp.clip(aarea + barea - inter, 1e-8)
    iou = inter / ua

    iou_max = jnp.max(iou, axis=0, keepdims=True)
    idx = jax.lax.broadcasted_iota(jnp.int32, (N, BLK), 0)
    arg = jnp.min(jnp.where(iou == iou_max, idx, N), axis=0, keepdims=True)
    onehotT = (idx == arg).astype(jnp.float32)

    posf = (iou_max >= 0.5).astype(jnp.float32)
    negf = (iou_max < 0.4).astype(jnp.float32)
    num_pos = jnp.sum(posf)

    p = jnp.clip(x, 1e-4, 1.0 - 1e-4)
    negterm = (p * p) * jnp.log(1.0 - p)
    w_row = negf + posf
    t1 = jax.lax.dot_general(w_row, negterm, (((1,), (0,)), ((), ())),
                             preferred_element_type=jnp.float32)
    cls_main = -0.75 * jnp.sum(t1)

    cidx = jax.lax.broadcasted_iota(jnp.int32, (C, N), 0)
    ohAT = (cidx == annT[20:21, :].astype(jnp.int32)).astype(jnp.float32)
    GT = jax.lax.dot_general(ohAT, p, (((0,), (1,)), ((), ())),
                             preferred_element_type=jnp.float32)
    pc = jnp.sum(onehotT * GT, axis=0, keepdims=True)
    fpos = 0.25 * (1.0 - pc) * (1.0 - pc) * (-jnp.log(pc))
    fneg = 0.75 * pc * pc * (-jnp.log(1.0 - pc))
    corr = jnp.sum(posf * (fpos - fneg))
    cls_blk = cls_main + corr

    predsT = jax.lax.dot_general(MT, r, (((1,), (1,)), ((), ())),
                                 preferred_element_type=jnp.float32)
    tT = jax.lax.dot_general(annT[0:20, :], onehotT, (((1,), (0,)), ((), ())),
                             preferred_element_type=jnp.float32)
    rowi = jax.lax.broadcasted_iota(jnp.int32, (20, BLK), 0)
    is_x = (rowi % 2) == 0
    inv_aw = 1.0 / aw
    inv_ah = 1.0 / ah
    t_norm = jnp.where(is_x, (tT - acx) * inv_aw, (tT - acy) * inv_ah)
    diff = jnp.abs(t_norm - predsT)
    rl = jnp.where(diff <= 1.0 / 9.0, 4.5 * diff * diff, diff - 0.5 / 9.0)
    reg_blk = jnp.sum(rl * posf)

    lane = jax.lax.broadcasted_iota(jnp.int32, (1, 1, 128), 2)
    vec = (jnp.where(lane == 0, cls_blk, 0.0)
           + jnp.where(lane == 1, reg_blk, 0.0)
           + jnp.where(lane == 2, num_pos, 0.0))

    @pl.when(j == 0)
    def _init():
        out_ref[...] = vec

    @pl.when(j != 0)
    def _accum():
        out_ref[...] = out_ref[...] + vec


def _pred_matrix_t() -> np.ndarray:
    m = np.zeros((12, 20), np.float32)
    for pt in range(8):
        s1 = 1.0 if pt & 1 else -1.0
        s2 = 1.0 if pt & 2 else -1.0
        s3 = 1.0 if pt & 4 else -1.0
        m[0, 2 * pt] = 1.0
        m[2, 2 * pt] = s1
        m[4, 2 * pt] = s2
        m[6, 2 * pt] = s3
        m[1, 2 * pt + 1] = 1.0
        m[3, 2 * pt + 1] = s1
        m[5, 2 * pt + 1] = s2
        m[7, 2 * pt + 1] = s3
    for k in range(4):
        m[8 + k, 16 + k] = 1.0
    return m.T.copy()


@functools.partial(jax.jit, static_argnames=("blk",))
def _run(classifications, regressions, anchors, annotations, blk):
    B, A, C = classifications.shape
    N = annotations.shape[1]
    nblk = A // blk
    annT = jnp.swapaxes(annotations, 1, 2)
    mT = jnp.asarray(_pred_matrix_t())

    out = pl.pallas_call(
        _focal_kernel,
        grid=(B, nblk),
        in_specs=[
            pl.BlockSpec((1, blk, C), lambda b, j: (b, j, 0)),
            pl.BlockSpec((1, blk, 12), lambda b, j: (b, j, 0)),
            pl.BlockSpec((1, blk, 4), lambda b, j: (0, j, 0)),
            pl.BlockSpec((1, N, 21), lambda b, j: (b, 0, 0)),
            pl.BlockSpec((1, 21, N), lambda b, j: (b, 0, 0)),
            pl.BlockSpec((20, 12), lambda b, j: (0, 0)),
        ],
        out_specs=pl.BlockSpec((1, 1, 128), lambda b, j: (b, 0, 0)),
        out_shape=jax.ShapeDtypeStruct((B, 1, 128), jnp.float32),
    )(classifications, regressions, anchors, annotations, annT, mT)

    cls_sum = out[:, 0, 0]
    reg_sum = out[:, 0, 1]
    npos = out[:, 0, 2]
    cls_total = cls_sum / jnp.maximum(npos, 1.0)
    reg_total = jnp.where(npos > 0.0,
                          reg_sum / (jnp.maximum(npos, 1.0) * 20.0), 0.0)
    return (jnp.mean(cls_total)[None], jnp.mean(reg_total)[None])


def kernel(classifications, regressions, anchors, annotations):
    A = classifications.shape[1]
    blk = 4000 if A % 4000 == 0 else A
    return _run(classifications, regressions, anchors, annotations, blk)


# speedup vs baseline: 1.2112x; 1.2108x over previous
import functools

import jax
import jax.numpy as jnp
import numpy as np
from jax.experimental import pallas as pl


def _focal_kernel(cls_ref, ra_ref, ann_ref, annT_ref, mT_ref,
                  out_ref):
    j = pl.program_id(1)

    x = cls_ref[0]
    ra = ra_ref[0, 0]
    rT = ra[0:12, :]
    anchT = ra[12:16, :]
    ann = ann_ref[0]
    annT = annT_ref[0]
    MT = mT_ref[...]

    BLK, C = x.shape
    N = ann.shape[0]

    def col(i):
        return ann[:, i:i + 1]

    xmin = jnp.minimum(jnp.minimum(col(0), col(2)), jnp.minimum(col(4), col(6)))
    xmax = jnp.maximum(jnp.maximum(col(0), col(2)), jnp.maximum(col(4), col(6)))
    ymin = jnp.minimum(jnp.minimum(col(1), col(3)), jnp.minimum(col(5), col(7)))
    ymax = jnp.maximum(jnp.maximum(col(1), col(3)), jnp.maximum(col(5), col(7)))
    xmin2 = jnp.minimum(jnp.minimum(col(8), col(10)), jnp.minimum(col(12), col(14)))
    xmax2 = jnp.maximum(jnp.maximum(col(8), col(10)), jnp.maximum(col(12), col(14)))
    ymin2 = jnp.minimum(jnp.minimum(col(9), col(11)), jnp.minimum(col(13), col(15)))
    ymax2 = jnp.maximum(jnp.maximum(col(9), col(11)), jnp.maximum(col(13), col(15)))
    bx1 = jnp.minimum(xmin, xmin2)
    by1 = jnp.minimum(ymin, ymin2)
    bx2 = jnp.maximum(xmax, xmax2)
    by2 = jnp.maximum(ymax, ymax2)
    barea = (bx2 - bx1) * (by2 - by1)

    ax1 = anchT[0:1, :]
    ay1 = anchT[1:2, :]
    ax2 = anchT[2:3, :]
    ay2 = anchT[3:4, :]
    aw = ax2 - ax1
    ah = ay2 - ay1
    acx = ax1 + 0.5 * aw
    acy = ay1 + 0.5 * ah
    aarea = aw * ah

    iw = jnp.clip(jnp.minimum(ax2, bx2) - jnp.maximum(ax1, bx1), 0.0)
    ih = jnp.clip(jnp.minimum(ay2, by2) - jnp.maximum(ay1, by1), 0.0)
    inter = iw * ih
    ua = jnp.clip(aarea + barea - inter, 1e-8)
    iou = inter / ua

    iou_max = jnp.max(iou, axis=0, keepdims=True)
    idx = jax.lax.broadcasted_iota(jnp.int32, (N, BLK), 0)
    arg = jnp.min(jnp.where(iou == iou_max, idx, N), axis=0, keepdims=True)
    onehotT = (idx == arg).astype(jnp.float32)

    posf = (iou_max >= 0.5).astype(jnp.float32)
    negf = (iou_max < 0.4).astype(jnp.float32)
    num_pos = jnp.sum(posf)

    p = jnp.clip(x, 1e-4, 1.0 - 1e-4)
    negterm = (p * p) * jnp.log(1.0 - p)
    w_row = negf + posf
    t1 = jax.lax.dot_general(w_row, negterm, (((1,), (0,)), ((), ())),
                             preferred_element_type=jnp.float32)
    cls_main = -0.75 * jnp.sum(t1)

    cidx = jax.lax.broadcasted_iota(jnp.int32, (C, N), 0)
    ohAT = (cidx == annT[20:21, :].astype(jnp.int32)).astype(jnp.float32)
    GT = jax.lax.dot_general(ohAT, p, (((0,), (1,)), ((), ())),
                             preferred_element_type=jnp.float32)
    pc = jnp.sum(onehotT * GT, axis=0, keepdims=True)
    fpos = 0.25 * (1.0 - pc) * (1.0 - pc) * (-jnp.log(pc))
    fneg = 0.75 * pc * pc * (-jnp.log(1.0 - pc))
    corr = jnp.sum(posf * (fpos - fneg))
    cls_blk = cls_main + corr

    predsT = jax.lax.dot_general(MT, rT, (((1,), (0,)), ((), ())),
                                 preferred_element_type=jnp.float32)
    tT = jax.lax.dot_general(annT[0:20, :], onehotT, (((1,), (0,)), ((), ())),
                             preferred_element_type=jnp.float32)
    rowi = jax.lax.broadcasted_iota(jnp.int32, (20, BLK), 0)
    is_x = (rowi % 2) == 0
    inv_aw = 1.0 / aw
    inv_ah = 1.0 / ah
    t_norm = jnp.where(is_x, (tT - acx) * inv_aw, (tT - acy) * inv_ah)
    diff = jnp.abs(t_norm - predsT)
    rl = jnp.where(diff <= 1.0 / 9.0, 4.5 * diff * diff, diff - 0.5 / 9.0)
    reg_blk = jnp.sum(rl * posf)

    lane = jax.lax.broadcasted_iota(jnp.int32, (1, 1, 128), 2)
    vec = (jnp.where(lane == 0, cls_blk, 0.0)
           + jnp.where(lane == 1, reg_blk, 0.0)
           + jnp.where(lane == 2, num_pos, 0.0))

    @pl.when(j == 0)
    def _init():
        out_ref[...] = vec

    @pl.when(j != 0)
    def _accum():
        out_ref[...] = out_ref[...] + vec


def _pred_matrix_t() -> np.ndarray:
    m = np.zeros((12, 20), np.float32)
    for pt in range(8):
        s1 = 1.0 if pt & 1 else -1.0
        s2 = 1.0 if pt & 2 else -1.0
        s3 = 1.0 if pt & 4 else -1.0
        m[0, 2 * pt] = 1.0
        m[2, 2 * pt] = s1
        m[4, 2 * pt] = s2
        m[6, 2 * pt] = s3
        m[1, 2 * pt + 1] = 1.0
        m[3, 2 * pt + 1] = s1
        m[5, 2 * pt + 1] = s2
        m[7, 2 * pt + 1] = s3
    for k in range(4):
        m[8 + k, 16 + k] = 1.0
    return m.T.copy()


@functools.partial(jax.jit, static_argnames=("blk",))
def _run(classifications, regressions, anchors, annotations, blk):
    B, A, C = classifications.shape
    N = annotations.shape[1]
    nblk = A // blk
    annT = jnp.swapaxes(annotations, 1, 2)
    mT = jnp.asarray(_pred_matrix_t())

    ra = jnp.concatenate(
        [regressions, jnp.broadcast_to(anchors, (B, A, 4))], axis=2)
    ra = jnp.swapaxes(ra, 1, 2).reshape(B, 16, nblk, blk)
    ra = jnp.swapaxes(ra, 1, 2)

    out = pl.pallas_call(
        _focal_kernel,
        grid=(B, nblk),
        in_specs=[
            pl.BlockSpec((1, blk, C), lambda b, j: (b, j, 0)),
            pl.BlockSpec((1, 1, 16, blk), lambda b, j: (b, j, 0, 0)),
            pl.BlockSpec((1, N, 21), lambda b, j: (b, 0, 0)),
            pl.BlockSpec((1, 21, N), lambda b, j: (b, 0, 0)),
            pl.BlockSpec((20, 12), lambda b, j: (0, 0)),
        ],
        out_specs=pl.BlockSpec((1, 1, 128), lambda b, j: (b, 0, 0)),
        out_shape=jax.ShapeDtypeStruct((B, 1, 128), jnp.float32),
    )(classifications, ra, annotations, annT, mT)

    cls_sum = out[:, 0, 0]
    reg_sum = out[:, 0, 1]
    npos = out[:, 0, 2]
    cls_total = cls_sum / jnp.maximum(npos, 1.0)
    reg_total = jnp.where(npos > 0.0,
                          reg_sum / (jnp.maximum(npos, 1.0) * 20.0), 0.0)
    return (jnp.mean(cls_total)[None], jnp.mean(reg_total)[None])


def kernel(classifications, regressions, anchors, annotations):
    A = classifications.shape[1]
    blk = 4000 if A % 4000 == 0 else A
    return _run(classifications, regressions, anchors, annotations, blk)


# speedup vs baseline: 1.3052x; 1.0776x over previous
import functools

import jax
import jax.numpy as jnp
import numpy as np
from jax.experimental import pallas as pl


def _focal_kernel(cls_ref, regT_ref, anchT_ref, ann_ref, annT_ref, mT_ref,
                  out_ref):
    j = pl.program_id(1)

    x = cls_ref[0]
    rT = regT_ref[0, 0]
    anchT = anchT_ref[0, 0]
    ann = ann_ref[0]
    annT = annT_ref[0]
    MT = mT_ref[...]

    BLK, C = x.shape
    N = ann.shape[0]

    def col(i):
        return ann[:, i:i + 1]

    xmin = jnp.minimum(jnp.minimum(col(0), col(2)), jnp.minimum(col(4), col(6)))
    xmax = jnp.maximum(jnp.maximum(col(0), col(2)), jnp.maximum(col(4), col(6)))
    ymin = jnp.minimum(jnp.minimum(col(1), col(3)), jnp.minimum(col(5), col(7)))
    ymax = jnp.maximum(jnp.maximum(col(1), col(3)), jnp.maximum(col(5), col(7)))
    xmin2 = jnp.minimum(jnp.minimum(col(8), col(10)), jnp.minimum(col(12), col(14)))
    xmax2 = jnp.maximum(jnp.maximum(col(8), col(10)), jnp.maximum(col(12), col(14)))
    ymin2 = jnp.minimum(jnp.minimum(col(9), col(11)), jnp.minimum(col(13), col(15)))
    ymax2 = jnp.maximum(jnp.maximum(col(9), col(11)), jnp.maximum(col(13), col(15)))
    bx1 = jnp.minimum(xmin, xmin2)
    by1 = jnp.minimum(ymin, ymin2)
    bx2 = jnp.maximum(xmax, xmax2)
    by2 = jnp.maximum(ymax, ymax2)
    barea = (bx2 - bx1) * (by2 - by1)

    ax1 = anchT[0:1, :]
    ay1 = anchT[1:2, :]
    ax2 = anchT[2:3, :]
    ay2 = anchT[3:4, :]
    aw = ax2 - ax1
    ah = ay2 - ay1
    acx = ax1 + 0.5 * aw
    acy = ay1 + 0.5 * ah
    aarea = aw * ah

    iw = jnp.clip(jnp.minimum(ax2, bx2) - jnp.maximum(ax1, bx1), 0.0)
    ih = jnp.clip(jnp.minimum(ay2, by2) - jnp.maximum(ay1, by1), 0.0)
    inter = iw * ih
    ua = jnp.clip(aarea + barea - inter, 1e-8)
    iou = inter / ua

    iou_max = jnp.max(iou, axis=0, keepdims=True)
    idx = jax.lax.broadcasted_iota(jnp.int32, (N, BLK), 0)
    arg = jnp.min(jnp.where(iou == iou_max, idx, N), axis=0, keepdims=True)
    onehotT = (idx == arg).astype(jnp.float32)

    posf = (iou_max >= 0.5).astype(jnp.float32)
    negf = (iou_max < 0.4).astype(jnp.float32)
    num_pos = jnp.sum(posf)

    p = jnp.clip(x, 1e-4, 1.0 - 1e-4)
    negterm = (p * p) * jnp.log(1.0 - p)
    w_row = negf + posf
    t1 = jax.lax.dot_general(w_row, negterm, (((1,), (0,)), ((), ())),
                             preferred_element_type=jnp.float32)
    cls_main = -0.75 * jnp.sum(t1)

    cidx = jax.lax.broadcasted_iota(jnp.int32, (C, N), 0)
    ohAT = (cidx == annT[20:21, :].astype(jnp.int32)).astype(jnp.float32)
    GT = jax.lax.dot_general(ohAT, p, (((0,), (1,)), ((), ())),
                             preferred_element_type=jnp.float32)
    pc = jnp.sum(onehotT * GT, axis=0, keepdims=True)
    fpos = 0.25 * (1.0 - pc) * (1.0 - pc) * (-jnp.log(pc))
    fneg = 0.75 * pc * pc * (-jnp.log(1.0 - pc))
    corr = jnp.sum(posf * (fpos - fneg))
    cls_blk = cls_main + corr

    predsT = jax.lax.dot_general(MT, rT, (((1,), (0,)), ((), ())),
                                 preferred_element_type=jnp.float32)
    tT = jax.lax.dot_general(annT[0:20, :], onehotT, (((1,), (0,)), ((), ())),
                             preferred_element_type=jnp.float32)
    rowi = jax.lax.broadcasted_iota(jnp.int32, (20, BLK), 0)
    is_x = (rowi % 2) == 0
    inv_aw = 1.0 / aw
    inv_ah = 1.0 / ah
    t_norm = jnp.where(is_x, (tT - acx) * inv_aw, (tT - acy) * inv_ah)
    diff = jnp.abs(t_norm - predsT)
    rl = jnp.where(diff <= 1.0 / 9.0, 4.5 * diff * diff, diff - 0.5 / 9.0)
    reg_blk = jnp.sum(rl * posf)

    lane = jax.lax.broadcasted_iota(jnp.int32, (1, 1, 128), 2)
    vec = (jnp.where(lane == 0, cls_blk, 0.0)
           + jnp.where(lane == 1, reg_blk, 0.0)
           + jnp.where(lane == 2, num_pos, 0.0))

    @pl.when(j == 0)
    def _init():
        out_ref[...] = vec

    @pl.when(j != 0)
    def _accum():
        out_ref[...] = out_ref[...] + vec


def _pred_matrix_t() -> np.ndarray:
    m = np.zeros((12, 20), np.float32)
    for pt in range(8):
        s1 = 1.0 if pt & 1 else -1.0
        s2 = 1.0 if pt & 2 else -1.0
        s3 = 1.0 if pt & 4 else -1.0
        m[0, 2 * pt] = 1.0
        m[2, 2 * pt] = s1
        m[4, 2 * pt] = s2
        m[6, 2 * pt] = s3
        m[1, 2 * pt + 1] = 1.0
        m[3, 2 * pt + 1] = s1
        m[5, 2 * pt + 1] = s2
        m[7, 2 * pt + 1] = s3
    for k in range(4):
        m[8 + k, 16 + k] = 1.0
    return m.T.copy()


@functools.partial(jax.jit, static_argnames=("blk",))
def _run(classifications, regressions, anchors, annotations, blk):
    B, A, C = classifications.shape
    N = annotations.shape[1]
    nblk = A // blk
    annT = jnp.swapaxes(annotations, 1, 2)
    mT = jnp.asarray(_pred_matrix_t())

    regT = jnp.swapaxes(regressions, 1, 2).reshape(B, 12, nblk, blk)
    regT = jnp.swapaxes(regT, 1, 2)
    anchT = jnp.swapaxes(anchors, 1, 2).reshape(1, 4, nblk, blk)
    anchT = jnp.swapaxes(anchT, 1, 2)

    out = pl.pallas_call(
        _focal_kernel,
        grid=(B, nblk),
        in_specs=[
            pl.BlockSpec((1, blk, C), lambda b, j: (b, j, 0)),
            pl.BlockSpec((1, 1, 12, blk), lambda b, j: (b, j, 0, 0)),
            pl.BlockSpec((1, 1, 4, blk), lambda b, j: (0, j, 0, 0)),
            pl.BlockSpec((1, N, 21), lambda b, j: (b, 0, 0)),
            pl.BlockSpec((1, 21, N), lambda b, j: (b, 0, 0)),
            pl.BlockSpec((20, 12), lambda b, j: (0, 0)),
        ],
        out_specs=pl.BlockSpec((1, 1, 128), lambda b, j: (b, 0, 0)),
        out_shape=jax.ShapeDtypeStruct((B, 1, 128), jnp.float32),
    )(classifications, regT, anchT, annotations, annT, mT)

    cls_sum = out[:, 0, 0]
    reg_sum = out[:, 0, 1]
    npos = out[:, 0, 2]
    cls_total = cls_sum / jnp.maximum(npos, 1.0)
    reg_total = jnp.where(npos > 0.0,
                          reg_sum / (jnp.maximum(npos, 1.0) * 20.0), 0.0)
    return (jnp.mean(cls_total)[None], jnp.mean(reg_total)[None])


def kernel(classifications, regressions, anchors, annotations):
    A = classifications.shape[1]
    blk = 4000 if A % 4000 == 0 else A
    return _run(classifications, regressions, anchors, annotations, blk)


# speedup vs baseline: 1.4040x; 1.0757x over previous
import functools

import jax
import jax.numpy as jnp
import numpy as np
from jax.experimental import pallas as pl


def _focal_kernel(cls_ref, regT_ref, anchT_ref, ann_ref, annT_ref, mT_ref,
                  out_ref):
    j = pl.program_id(1)

    x = cls_ref[0]
    rT = regT_ref[0, 0]
    anchT = anchT_ref[0, 0]
    ann = ann_ref[0]
    annT = annT_ref[0]
    MT = mT_ref[...]

    BLK, C = x.shape
    N = ann.shape[0]

    def col(i):
        return ann[:, i:i + 1]

    xmin = jnp.minimum(jnp.minimum(col(0), col(2)), jnp.minimum(col(4), col(6)))
    xmax = jnp.maximum(jnp.maximum(col(0), col(2)), jnp.maximum(col(4), col(6)))
    ymin = jnp.minimum(jnp.minimum(col(1), col(3)), jnp.minimum(col(5), col(7)))
    ymax = jnp.maximum(jnp.maximum(col(1), col(3)), jnp.maximum(col(5), col(7)))
    xmin2 = jnp.minimum(jnp.minimum(col(8), col(10)), jnp.minimum(col(12), col(14)))
    xmax2 = jnp.maximum(jnp.maximum(col(8), col(10)), jnp.maximum(col(12), col(14)))
    ymin2 = jnp.minimum(jnp.minimum(col(9), col(11)), jnp.minimum(col(13), col(15)))
    ymax2 = jnp.maximum(jnp.maximum(col(9), col(11)), jnp.maximum(col(13), col(15)))
    bx1 = jnp.minimum(xmin, xmin2)
    by1 = jnp.minimum(ymin, ymin2)
    bx2 = jnp.maximum(xmax, xmax2)
    by2 = jnp.maximum(ymax, ymax2)
    barea = (bx2 - bx1) * (by2 - by1)

    ax1 = anchT[0:1, :]
    ay1 = anchT[1:2, :]
    ax2 = anchT[2:3, :]
    ay2 = anchT[3:4, :]
    aw = ax2 - ax1
    ah = ay2 - ay1
    acx = ax1 + 0.5 * aw
    acy = ay1 + 0.5 * ah
    aarea = aw * ah

    iw = jnp.clip(jnp.minimum(ax2, bx2) - jnp.maximum(ax1, bx1), 0.0)
    ih = jnp.clip(jnp.minimum(ay2, by2) - jnp.maximum(ay1, by1), 0.0)
    inter = iw * ih
    ua = jnp.clip(aarea + barea - inter, 1e-8)
    iou = inter / ua

    iou_max = jnp.max(iou, axis=0, keepdims=True)
    idx = jax.lax.broadcasted_iota(jnp.int32, (N, BLK), 0)
    arg = jnp.min(jnp.where(iou == iou_max, idx, N), axis=0, keepdims=True)
    onehotT = (idx == arg).astype(jnp.float32)

    posf = (iou_max >= 0.5).astype(jnp.float32)
    negf = (iou_max < 0.4).astype(jnp.float32)
    num_pos = jnp.sum(posf)

    p = jnp.clip(x, 1e-4, 1.0 - 1e-4)
    negterm = (p * p) * jnp.log(1.0 - p)
    w_row = negf + posf
    t1 = jax.lax.dot_general(w_row, negterm, (((1,), (0,)), ((), ())),
                             preferred_element_type=jnp.float32)
    cls_main = -0.75 * jnp.sum(t1)

    cidx = jax.lax.broadcasted_iota(jnp.int32, (C, N), 0)
    ohAT = (cidx == annT[20:21, :].astype(jnp.int32)).astype(jnp.float32)
    GT = jax.lax.dot_general(ohAT, p, (((0,), (1,)), ((), ())),
                             preferred_element_type=jnp.float32)
    pc = jnp.sum(onehotT * GT, axis=0, keepdims=True)
    fpos = 0.25 * (1.0 - pc) * (1.0 - pc) * (-jnp.log(pc))
    fneg = 0.75 * pc * pc * (-jnp.log(1.0 - pc))
    corr = jnp.sum(posf * (fpos - fneg))
    cls_blk = cls_main + corr

    predsT = jax.lax.dot_general(MT, rT, (((1,), (0,)), ((), ())),
                                 preferred_element_type=jnp.float32)
    tT = jax.lax.dot_general(annT[0:20, :], onehotT, (((1,), (0,)), ((), ())),
                             preferred_element_type=jnp.float32)
    rowi = jax.lax.broadcasted_iota(jnp.int32, (20, BLK), 0)
    is_x = (rowi % 2) == 0
    inv_aw = 1.0 / aw
    inv_ah = 1.0 / ah
    t_norm = jnp.where(is_x, (tT - acx) * inv_aw, (tT - acy) * inv_ah)
    diff = jnp.abs(t_norm - predsT)
    rl = jnp.where(diff <= 1.0 / 9.0, 4.5 * diff * diff, diff - 0.5 / 9.0)
    reg_blk = jnp.sum(rl * posf)

    lane = jax.lax.broadcasted_iota(jnp.int32, (1, 1, 128), 2)
    vec = (jnp.where(lane == 0, cls_blk, 0.0)
           + jnp.where(lane == 1, reg_blk, 0.0)
           + jnp.where(lane == 2, num_pos, 0.0))

    @pl.when(j == 0)
    def _init():
        out_ref[...] = vec

    @pl.when(j != 0)
    def _accum():
        out_ref[...] = out_ref[...] + vec


def _pred_matrix_t() -> np.ndarray:
    m = np.zeros((12, 20), np.float32)
    for pt in range(8):
        s1 = 1.0 if pt & 1 else -1.0
        s2 = 1.0 if pt & 2 else -1.0
        s3 = 1.0 if pt & 4 else -1.0
        m[0, 2 * pt] = 1.0
        m[2, 2 * pt] = s1
        m[4, 2 * pt] = s2
        m[6, 2 * pt] = s3
        m[1, 2 * pt + 1] = 1.0
        m[3, 2 * pt + 1] = s1
        m[5, 2 * pt + 1] = s2
        m[7, 2 * pt + 1] = s3
    for k in range(4):
        m[8 + k, 16 + k] = 1.0
    return m.T.copy()


@functools.partial(jax.jit, static_argnames=("blk",))
def _run(classifications, regressions, anchors, annotations, blk):
    B, A, C = classifications.shape
    N = annotations.shape[1]
    nblk = A // blk
    annT = jnp.swapaxes(annotations, 1, 2)
    mT = jnp.asarray(_pred_matrix_t())

    regT = jnp.swapaxes(regressions, 1, 2).reshape(B, 12, nblk, blk)
    regT = jnp.swapaxes(regT, 1, 2)
    anchT = jnp.swapaxes(anchors, 1, 2).reshape(1, 4, nblk, blk)
    anchT = jnp.swapaxes(anchT, 1, 2)

    out = pl.pallas_call(
        _focal_kernel,
        grid=(B, nblk),
        in_specs=[
            pl.BlockSpec((1, blk, C), lambda b, j: (b, j, 0)),
            pl.BlockSpec((1, 1, 12, blk), lambda b, j: (b, j, 0, 0)),
            pl.BlockSpec((1, 1, 4, blk), lambda b, j: (0, j, 0, 0)),
            pl.BlockSpec((1, N, 21), lambda b, j: (b, 0, 0)),
            pl.BlockSpec((1, 21, N), lambda b, j: (b, 0, 0)),
            pl.BlockSpec((20, 12), lambda b, j: (0, 0)),
        ],
        out_specs=pl.BlockSpec((1, 1, 128), lambda b, j: (b, 0, 0)),
        out_shape=jax.ShapeDtypeStruct((B, 1, 128), jnp.float32),
    )(classifications, regT, anchT, annotations, annT, mT)

    cls_sum = out[:, 0, 0]
    reg_sum = out[:, 0, 1]
    npos = out[:, 0, 2]
    cls_total = cls_sum / jnp.maximum(npos, 1.0)
    reg_total = jnp.where(npos > 0.0,
                          reg_sum / (jnp.maximum(npos, 1.0) * 20.0), 0.0)
    return (jnp.mean(cls_total)[None], jnp.mean(reg_total)[None])


def kernel(classifications, regressions, anchors, annotations):
    A = classifications.shape[1]
    blk = 10000 if A % 10000 == 0 else A
    return _run(classifications, regressions, anchors, annotations, blk)


# speedup vs baseline: 1.4204x; 1.0117x over previous
import functools

import jax
import jax.numpy as jnp
import numpy as np
from jax.experimental import pallas as pl


def _focal_kernel(cls_ref, regT_ref, anchT_ref, ann_ref, annT_ref, mT_ref,
                  out_ref):
    j = pl.program_id(1)

    x = cls_ref[0]
    rT = regT_ref[0, 0]
    anchT = anchT_ref[0, 0]
    ann = ann_ref[0]
    annT = annT_ref[0]
    MT = mT_ref[...]

    BLK, C = x.shape
    N = ann.shape[0]

    def col(i):
        return ann[:, i:i + 1]

    xmin = jnp.minimum(jnp.minimum(col(0), col(2)), jnp.minimum(col(4), col(6)))
    xmax = jnp.maximum(jnp.maximum(col(0), col(2)), jnp.maximum(col(4), col(6)))
    ymin = jnp.minimum(jnp.minimum(col(1), col(3)), jnp.minimum(col(5), col(7)))
    ymax = jnp.maximum(jnp.maximum(col(1), col(3)), jnp.maximum(col(5), col(7)))
    xmin2 = jnp.minimum(jnp.minimum(col(8), col(10)), jnp.minimum(col(12), col(14)))
    xmax2 = jnp.maximum(jnp.maximum(col(8), col(10)), jnp.maximum(col(12), col(14)))
    ymin2 = jnp.minimum(jnp.minimum(col(9), col(11)), jnp.minimum(col(13), col(15)))
    ymax2 = jnp.maximum(jnp.maximum(col(9), col(11)), jnp.maximum(col(13), col(15)))
    bx1 = jnp.minimum(xmin, xmin2)
    by1 = jnp.minimum(ymin, ymin2)
    bx2 = jnp.maximum(xmax, xmax2)
    by2 = jnp.maximum(ymax, ymax2)
    barea = (bx2 - bx1) * (by2 - by1)

    ax1 = anchT[0:1, :]
    ay1 = anchT[1:2, :]
    ax2 = anchT[2:3, :]
    ay2 = anchT[3:4, :]
    aw = ax2 - ax1
    ah = ay2 - ay1
    acx = ax1 + 0.5 * aw
    acy = ay1 + 0.5 * ah
    aarea = aw * ah

    iw = jnp.clip(jnp.minimum(ax2, bx2) - jnp.maximum(ax1, bx1), 0.0)
    ih = jnp.clip(jnp.minimum(ay2, by2) - jnp.maximum(ay1, by1), 0.0)
    inter = iw * ih
    ua = jnp.clip(aarea + barea - inter, 1e-8)
    iou = inter / ua

    iou_max = jnp.max(iou, axis=0, keepdims=True)
    idx = jax.lax.broadcasted_iota(jnp.int32, (N, BLK), 0)
    arg = jnp.min(jnp.where(iou == iou_max, idx, N), axis=0, keepdims=True)
    onehotT = (idx == arg).astype(jnp.float32)

    posf = (iou_max >= 0.5).astype(jnp.float32)
    negf = (iou_max < 0.4).astype(jnp.float32)
    num_pos = jnp.sum(posf)

    p = jnp.clip(x, 1e-4, 1.0 - 1e-4)
    negterm = (p * p) * jnp.log(1.0 - p)
    w_row = negf + posf
    t1 = jax.lax.dot_general(w_row, negterm, (((1,), (0,)), ((), ())),
                             preferred_element_type=jnp.float32)
    cls_main = -0.75 * jnp.sum(t1)

    cidx = jax.lax.broadcasted_iota(jnp.int32, (C, N), 0)
    ohAT = (cidx == annT[20:21, :].astype(jnp.int32)).astype(jnp.float32)
    GT = jax.lax.dot_general(ohAT, p, (((0,), (1,)), ((), ())),
                             preferred_element_type=jnp.float32)
    pc = jnp.sum(onehotT * GT, axis=0, keepdims=True)
    fpos = 0.25 * (1.0 - pc) * (1.0 - pc) * (-jnp.log(pc))
    fneg = 0.75 * pc * pc * (-jnp.log(1.0 - pc))
    corr = jnp.sum(posf * (fpos - fneg))
    cls_blk = cls_main + corr

    predsT = jax.lax.dot_general(MT, rT, (((1,), (0,)), ((), ())),
                                 preferred_element_type=jnp.float32)
    tT = jax.lax.dot_general(annT[0:20, :], onehotT, (((1,), (0,)), ((), ())),
                             preferred_element_type=jnp.float32)
    rowi = jax.lax.broadcasted_iota(jnp.int32, (20, BLK), 0)
    is_x = (rowi % 2) == 0
    inv_aw = 1.0 / aw
    inv_ah = 1.0 / ah
    t_norm = jnp.where(is_x, (tT - acx) * inv_aw, (tT - acy) * inv_ah)
    diff = jnp.abs(t_norm - predsT)
    rl = jnp.where(diff <= 1.0 / 9.0, 4.5 * diff * diff, diff - 0.5 / 9.0)
    reg_blk = jnp.sum(rl * posf)

    lane = jax.lax.broadcasted_iota(jnp.int32, (1, 1, 128), 2)
    vec = (jnp.where(lane == 0, cls_blk, 0.0)
           + jnp.where(lane == 1, reg_blk, 0.0)
           + jnp.where(lane == 2, num_pos, 0.0))

    @pl.when(j == 0)
    def _init():
        out_ref[...] = vec

    @pl.when(j != 0)
    def _accum():
        out_ref[...] = out_ref[...] + vec


def _pred_matrix_t() -> np.ndarray:
    m = np.zeros((12, 20), np.float32)
    for pt in range(8):
        s1 = 1.0 if pt & 1 else -1.0
        s2 = 1.0 if pt & 2 else -1.0
        s3 = 1.0 if pt & 4 else -1.0
        m[0, 2 * pt] = 1.0
        m[2, 2 * pt] = s1
        m[4, 2 * pt] = s2
        m[6, 2 * pt] = s3
        m[1, 2 * pt + 1] = 1.0
        m[3, 2 * pt + 1] = s1
        m[5, 2 * pt + 1] = s2
        m[7, 2 * pt + 1] = s3
    for k in range(4):
        m[8 + k, 16 + k] = 1.0
    return m.T.copy()


@functools.partial(jax.jit, static_argnames=("blk",))
def _run(classifications, regressions, anchors, annotations, blk):
    B, A, C = classifications.shape
    N = annotations.shape[1]
    nblk = A // blk
    annT = jnp.swapaxes(annotations, 1, 2)
    mT = jnp.asarray(_pred_matrix_t())

    regT = jnp.swapaxes(regressions, 1, 2).reshape(B, 12, nblk, blk)
    regT = jnp.swapaxes(regT, 1, 2)
    anchT = jnp.swapaxes(anchors, 1, 2).reshape(1, 4, nblk, blk)
    anchT = jnp.swapaxes(anchT, 1, 2)

    out = pl.pallas_call(
        _focal_kernel,
        grid=(B, nblk),
        in_specs=[
            pl.BlockSpec((1, blk, C), lambda b, j: (b, j, 0)),
            pl.BlockSpec((1, 1, 12, blk), lambda b, j: (b, j, 0, 0)),
            pl.BlockSpec((1, 1, 4, blk), lambda b, j: (0, j, 0, 0)),
            pl.BlockSpec((1, N, 21), lambda b, j: (b, 0, 0)),
            pl.BlockSpec((1, 21, N), lambda b, j: (b, 0, 0)),
            pl.BlockSpec((20, 12), lambda b, j: (0, 0)),
        ],
        out_specs=pl.BlockSpec((1, 1, 128), lambda b, j: (b, 0, 0)),
        out_shape=jax.ShapeDtypeStruct((B, 1, 128), jnp.float32),
    )(classifications, regT, anchT, annotations, annT, mT)

    cls_sum = out[:, 0, 0]
    reg_sum = out[:, 0, 1]
    npos = out[:, 0, 2]
    cls_total = cls_sum / jnp.maximum(npos, 1.0)
    reg_total = jnp.where(npos > 0.0,
                          reg_sum / (jnp.maximum(npos, 1.0) * 20.0), 0.0)
    return (jnp.mean(cls_total)[None], jnp.mean(reg_total)[None])


def kernel(classifications, regressions, anchors, annotations):
    A = classifications.shape[1]
    blk = 20000 if A % 20000 == 0 else A
    return _run(classifications, regressions, anchors, annotations, blk)
